# Initial kernel scaffold; baseline (speedup 1.0000x reference)
import jax
import jax.numpy as jnp
from jax.experimental import pallas as pl


def kernel(x, func_parameter):
    raise NotImplementedError("write your pallas kernel here")


# speedup vs baseline: 17.0574x; 17.0574x over previous
import functools
import math

import numpy as np
import jax
import jax.numpy as jnp
from jax import lax
from jax.experimental import pallas as pl
from jax.experimental.pallas import tpu as pltpu
from jax.experimental.pallas import tpu_sc as plsc

_NUM_CORES = 2
_NUM_SUBCORES = 16
_NW = _NUM_CORES * _NUM_SUBCORES


def _host_borders(n_chunks: int) -> np.ndarray:
    def inverse_grid_function(v):
        if v <= 0.5:
            return math.log(2.0 * v)
        return -math.log(2.0 * (1.0 - v))

    chunk_size = 1.0 / n_chunks
    borders = [inverse_grid_function(i * chunk_size) for i in range(1, n_chunks)]
    left_most = borders[0] - (borders[1] - borders[0])
    right_most = borders[-1] + (borders[-1] - borders[-2])
    return np.array([left_most] + borders + [right_most], dtype=np.float32)


@functools.lru_cache(maxsize=None)
def _build_sc_call(in_dim: int, batch: int, g1: int, od: int):
    P = in_dim // 2
    G = g1 - 1
    BPW = batch // _NW
    HALF = BPW // 2
    NCH = HALF // 16
    OC = od // 16

    mesh = plsc.VectorSubcoreMesh(
        core_axis_name="c", subcore_axis_name="s",
        num_cores=_NUM_CORES, num_subcores=_NUM_SUBCORES)

    @functools.partial(
        pl.kernel,
        out_type=jax.ShapeDtypeStruct((batch, od), jnp.float32),
        mesh=mesh,
        scratch_types=[
            pltpu.VMEM((in_dim, BPW), jnp.float32),
            pltpu.VMEM((g1,), jnp.float32),
            pltpu.VMEM((G,), jnp.float32),
            pltpu.VMEM((2, 4, HALF), jnp.int32),
            pltpu.VMEM((2, 4, HALF), jnp.float32),
            pltpu.VMEM((2, 4, HALF, od), jnp.float32),
            pltpu.VMEM((BPW, od), jnp.float32),
            pltpu.SemaphoreType.DMA,
            pltpu.SemaphoreType.DMA,
        ],
        compiler_params=pltpu.CompilerParams(needs_layout_passes=False),
    )
    def sc_call(x_hbm, table_hbm, bord_hbm, inv_hbm, out_hbm,
                xv, bord_v, inv_v, idx_v, w_v, rows_v, acc_v, sem0, sem1):
        wid = lax.axis_index("s") * _NUM_CORES + lax.axis_index("c")
        base = wid * BPW

        pltpu.sync_copy(x_hbm.at[:, pl.ds(base, BPW)], xv)
        pltpu.sync_copy(bord_hbm, bord_v)
        pltpu.sync_copy(inv_hbm, inv_v)

        zeros = jnp.zeros((16,), jnp.float32)

        def zrow(i, _):
            for oc in range(OC):
                acc_v[i, pl.ds(oc * 16, 16)] = zeros
            return 0
        lax.fori_loop(0, BPW, zrow, 0)

        def compute_issue(slot, p, h):
            sem = sem0 if slot == 0 else sem1
            for j in range(NCH):
                col = h * HALF + j * 16
                x1 = xv[2 * p, pl.ds(col, 16)]
                x2 = xv[2 * p + 1, pl.ds(col, 16)]
                e1 = jnp.exp(-jnp.abs(x1))
                e2 = jnp.exp(-jnp.abs(x2))
                c1 = jnp.where(x1 > 0, 1.0 - 0.5 * e1, 0.5 * e1)
                c2 = jnp.where(x2 > 0, 1.0 - 0.5 * e2, 0.5 * e2)
                i1 = jnp.clip((c1 * float(G)).astype(jnp.int32), 0, G - 1)
                i2 = jnp.clip((c2 * float(G)).astype(jnp.int32), 0, G - 1)
                l1 = plsc.load_gather(bord_v, [i1])
                l2 = plsc.load_gather(bord_v, [i2])
                v1 = plsc.load_gather(inv_v, [i1])
                v2 = plsc.load_gather(inv_v, [i2])
                d1 = (x1 - l1) * v1
                d2 = (x2 - l2) * v2
                row = (p * g1 + i1) * g1 + i2
                sl = pl.ds(j * 16, 16)
                idx_v[slot, 0, sl] = row
                idx_v[slot, 1, sl] = row + 1
                idx_v[slot, 2, sl] = row + g1
                idx_v[slot, 3, sl] = row + g1 + 1
                om1 = 1.0 - d1
                om2 = 1.0 - d2
                w_v[slot, 0, sl] = om1 * om2
                w_v[slot, 1, sl] = om1 * d2
                w_v[slot, 2, sl] = d1 * om2
                w_v[slot, 3, sl] = d1 * d2
            for c in range(4):
                pltpu.async_copy(
                    table_hbm.at[idx_v.at[slot, c]], rows_v.at[slot, c], sem)

        def wait_gathers(slot):
            sem = sem0 if slot == 0 else sem1
            for c in range(4):
                pltpu.make_async_copy(
                    table_hbm.at[idx_v.at[slot, c]], rows_v.at[slot, c], sem
                ).wait()

        def accumulate(slot, h):
            slot_v = jnp.full((16,), slot, jnp.int32)

            def bbody(b, _):
                arow = h * HALF + b
                b_v = jnp.full((16,), b, jnp.int32)
                w0 = plsc.load_gather(w_v, [slot_v, jnp.full((16,), 0, jnp.int32), b_v])
                w1 = plsc.load_gather(w_v, [slot_v, jnp.full((16,), 1, jnp.int32), b_v])
                w2 = plsc.load_gather(w_v, [slot_v, jnp.full((16,), 2, jnp.int32), b_v])
                w3 = plsc.load_gather(w_v, [slot_v, jnp.full((16,), 3, jnp.int32), b_v])
                for oc in range(OC):
                    sl = pl.ds(oc * 16, 16)
                    a = acc_v[arow, sl]
                    a = a + w0 * rows_v[slot, 0, b, sl]
                    a = a + w1 * rows_v[slot, 1, b, sl]
                    a = a + w2 * rows_v[slot, 2, b, sl]
                    a = a + w3 * rows_v[slot, 3, b, sl]
                    acc_v[arow, sl] = a
                return 0
            lax.fori_loop(0, HALF, bbody, 0)

        compute_issue(0, 0, 0)

        def pair_body(k, _):
            compute_issue(1, k, 1)
            wait_gathers(0)
            accumulate(0, 0)

            @pl.when(k < P - 1)
            def _():
                compute_issue(0, k + 1, 0)

            wait_gathers(1)
            accumulate(1, 1)
            return 0
        lax.fori_loop(0, P, pair_body, 0)

        pltpu.sync_copy(acc_v, out_hbm.at[pl.ds(base, BPW), :])

    return sc_call


def kernel(x, func_parameter):
    in_dim, batch = x.shape
    g1, _, od, n_pairs = func_parameter.shape
    G = g1 - 1
    table = jnp.transpose(func_parameter, (3, 0, 1, 2)).reshape(
        n_pairs * g1 * g1, od)
    borders_np = _host_borders(G)
    inv_np = (1.0 / (borders_np[1:] - borders_np[:-1])).astype(np.float32)
    sc_call = _build_sc_call(in_dim, batch, g1, od)
    out = sc_call(x, table, jnp.asarray(borders_np), jnp.asarray(inv_np))
    return out.T
